# Initial kernel scaffold; baseline (speedup 1.0000x reference)
import jax
import jax.numpy as jnp
from jax.experimental import pallas as pl


def kernel(h, e, edge_index):
    raise NotImplementedError("write your pallas kernel here")


# speedup vs baseline: 4.7569x; 4.7569x over previous
import functools

import jax
import jax.numpy as jnp
from jax import lax
from jax.experimental import pallas as pl
from jax.experimental.pallas import tpu as pltpu
from jax.experimental.pallas import tpu_sc as plsc

NC = 2
NS = 16
NW = NC * NS

_W0 = 0.3535533905932738
_W1 = 0.6123724356957945

_CHUNK = 128


@functools.lru_cache(maxsize=None)
def _make_segsum(n, ev, d):
    assert ev % _CHUNK == 0
    total_chunks = ev // _CHUNK
    base_ch = total_chunks // NW
    extra = total_chunks % NW
    np_ = ((n + NS * 8 - 1) // (NS * 8)) * (NS * 8)
    rows_pt = np_ // NS
    assert d % 16 == 0

    mesh = plsc.VectorSubcoreMesh(core_axis_name="c", subcore_axis_name="s")

    @functools.partial(
        pl.kernel,
        out_type=[
            jax.ShapeDtypeStruct((np_, d), jnp.float32),
            jax.ShapeDtypeStruct((np_, d), jnp.float32),
        ],
        mesh=mesh,
        scratch_types=[
            pltpu.VMEM_SHARED((np_, d), jnp.float32),
            pltpu.VMEM((_CHUNK,), jnp.int32),
            pltpu.VMEM((_CHUNK,), jnp.int32),
            pltpu.VMEM((_CHUNK,), jnp.float32),
            pltpu.VMEM((_CHUNK, d), jnp.float32),
            pltpu.SemaphoreType.DMA,
        ],
    )
    def segsum(x_hbm, src_hbm, dst_hbm, e_hbm, z_hbm,
               out0, out1, acc, src_v, dst_v, e_v, rows_v, sem):
        c = lax.axis_index("c")
        s = lax.axis_index("s")
        wid = s * NC + c
        r0 = s * rows_pt
        pltpu.sync_copy(z_hbm.at[pl.ds(r0, rows_pt)], acc.at[pl.ds(r0, rows_pt)])
        plsc.subcore_barrier()

        n_ch = base_ch + jnp.where(wid < extra, 1, 0)

        def chunk_body(ci, _):
            base = (wid + ci * NW) * _CHUNK
            pltpu.sync_copy(src_hbm.at[pl.ds(base, _CHUNK)], src_v)
            pltpu.sync_copy(dst_hbm.at[pl.ds(base, _CHUNK)], dst_v)
            pltpu.sync_copy(e_hbm.at[pl.ds(base, _CHUNK)], e_v)
            pltpu.async_copy(x_hbm.at[src_v], rows_v, sem).wait()

            def group_body(g, _):
                gsl = pl.ds(g * 16, 16)
                wv = jnp.where(src_v[gsl] == dst_v[gsl],
                               jnp.float32(-1.0), e_v[gsl])
                for j in range(16):
                    wsp = jnp.full((16,), wv[j], jnp.float32)
                    row = g * 16 + j
                    for q in range(d // 16):
                        sl = pl.ds(q * 16, 16)
                        rows_v[row, sl] = rows_v[row, sl] * wsp
                return 0

            lax.fori_loop(0, _CHUNK // 16, group_body, 0)
            pltpu.sync_copy(rows_v, acc.at[dst_v], add=True)
            return 0

        lax.fori_loop(0, n_ch, chunk_body, 0)
        plsc.subcore_barrier()

        @pl.when(c == 0)
        def _():
            pltpu.sync_copy(acc.at[pl.ds(r0, rows_pt)], out0.at[pl.ds(r0, rows_pt)])

        @pl.when(c == 1)
        def _():
            pltpu.sync_copy(acc.at[pl.ds(r0, rows_pt)], out1.at[pl.ds(r0, rows_pt)])

    return segsum


def _row_grid(n, d, n_in, body, out_shape=None, block_rows=2000):
    grid = (pl.cdiv(n, block_rows),)
    in_specs = [pl.BlockSpec((block_rows, d), lambda i: (i, 0))] * n_in
    if out_shape is None:
        out_shape = (n, d)
    out_spec = pl.BlockSpec((block_rows, out_shape[1]), lambda i: (i, 0))
    return pl.pallas_call(
        body,
        grid=grid,
        in_specs=in_specs,
        out_specs=out_spec,
        out_shape=jax.ShapeDtypeStruct(out_shape, jnp.float32),
    )


def _x1_body(x0, a00, a01, o):
    o[...] = x0[...] + (a00[...] + a01[...]) * jnp.float32(1.0 / 3.0)


def _x2_body(x0, a00, a01, a10, a11, o):
    o[...] = x0[...] + (a10[...] + a11[...]) - (a00[...] + a01[...]) * jnp.float32(1.0 / 3.0)


def _g_body(x0, x1, x2, a00, a01, a10, a11, a20, a21, o):
    x3 = (x0[...] + (a00[...] + a01[...]) - (a10[...] + a11[...])
          + (a20[...] + a21[...]))
    d = x0.shape[1]
    o[:, 0 * d:1 * d] = x0[...] * jnp.float32(_W0)
    o[:, 1 * d:2 * d] = x1[...] * jnp.float32(_W1)
    o[:, 2 * d:3 * d] = x2[...] * jnp.float32(_W1)
    o[:, 3 * d:4 * d] = x3 * jnp.float32(_W0)


def _mm_body(gi, gj, o):
    o[...] = lax.dot_general(
        gi[...], gj[...], (((1,), (1,)), ((), ())),
        preferred_element_type=jnp.float32,
    )


@functools.lru_cache(maxsize=None)
def _make_gram(n, k, bm, bn):
    grid = (pl.cdiv(n, bm), pl.cdiv(n, bn))
    return pl.pallas_call(
        _mm_body,
        grid=grid,
        in_specs=[
            pl.BlockSpec((bm, k), lambda i, j: (i, 0)),
            pl.BlockSpec((bn, k), lambda i, j: (j, 0)),
        ],
        out_specs=pl.BlockSpec((bm, bn), lambda i, j: (i, j)),
        out_shape=jax.ShapeDtypeStruct((n, n), jnp.float32),
        compiler_params=pltpu.CompilerParams(
            dimension_semantics=("parallel", "parallel"),
        ),
    )


def kernel(h, e, edge_index):
    n, d = h.shape
    ev = e.shape[0]
    src = edge_index[0]
    dst = edge_index[1]
    ew = e.reshape(ev)
    np_ = ((n + NS * 8 - 1) // (NS * 8)) * (NS * 8)
    zeros = jnp.zeros((np_, d), jnp.float32)

    segsum = _make_segsum(n, ev, d)
    a00, a01 = segsum(h, src, dst, ew, zeros)
    x1 = _row_grid(n, d, 3, _x1_body)(h, a00, a01)
    a10, a11 = segsum(x1, src, dst, ew, zeros)
    x2 = _row_grid(n, d, 5, _x2_body)(h, a00, a01, a10, a11)
    a20, a21 = segsum(x2, src, dst, ew, zeros)
    g = _row_grid(n, d, 9, _g_body, out_shape=(n, 4 * d))(
        h, x1, x2, a00, a01, a10, a11, a20, a21)
    return _make_gram(n, 4 * d, 1024, 1024)(g, g)


# speedup vs baseline: 7.8160x; 1.6431x over previous
import functools

import jax
import jax.numpy as jnp
from jax import lax
from jax.experimental import pallas as pl
from jax.experimental.pallas import tpu as pltpu
from jax.experimental.pallas import tpu_sc as plsc

NC = 2
NS = 16
NW = NC * NS

_W0 = 0.3535533905932738
_W1 = 0.6123724356957945

_CHUNK = 128


@functools.lru_cache(maxsize=None)
def _make_segsum(n, tch, d):
    assert tch % NW == 0
    cpt = tch // NW
    np_ = ((n + NS * 8 - 1) // (NS * 8)) * (NS * 8)
    rows_pt = np_ // NS
    assert d % 16 == 0
    ngr = _CHUNK // 16

    mesh = plsc.VectorSubcoreMesh(core_axis_name="c", subcore_axis_name="s")

    @functools.partial(
        pl.kernel,
        out_type=[
            jax.ShapeDtypeStruct((np_, d), jnp.float32),
            jax.ShapeDtypeStruct((np_, d), jnp.float32),
        ],
        mesh=mesh,
        scratch_types=[
            pltpu.VMEM_SHARED((np_, d), jnp.float32),
            pltpu.VMEM((cpt, _CHUNK), jnp.int32),
            pltpu.VMEM((cpt, _CHUNK), jnp.int32),
            pltpu.VMEM((cpt, _CHUNK), jnp.float32),
            pltpu.VMEM((2, _CHUNK, d), jnp.float32),
            pltpu.SemaphoreType.DMA,
            pltpu.SemaphoreType.DMA,
            pltpu.SemaphoreType.DMA,
            pltpu.SemaphoreType.DMA,
        ],
    )
    def segsum(x_hbm, src_hbm, dst_hbm, e_hbm, z_hbm,
               out0, out1, acc, src_v, dst_v, w_v, rows_v,
               gsem0, gsem1, ssem0, ssem1):
        c = lax.axis_index("c")
        s = lax.axis_index("s")
        wid = s * NC + c
        r0 = s * rows_pt
        pltpu.sync_copy(z_hbm.at[pl.ds(r0, rows_pt)], acc.at[pl.ds(r0, rows_pt)])
        cb = wid * cpt
        pltpu.sync_copy(src_hbm.at[pl.ds(cb, cpt)], src_v)
        pltpu.sync_copy(dst_hbm.at[pl.ds(cb, cpt)], dst_v)
        pltpu.sync_copy(e_hbm.at[pl.ds(cb, cpt)], w_v)

        def wbody(ci, _):
            for g in range(ngr):
                sl = pl.ds(g * 16, 16)
                w_v[ci, sl] = jnp.where(src_v[ci, sl] == dst_v[ci, sl],
                                        jnp.float32(-1.0), w_v[ci, sl])
            return 0

        lax.fori_loop(0, cpt, wbody, 0)
        plsc.subcore_barrier()

        gsems = (gsem0, gsem1)
        ssems = (ssem0, ssem1)

        def gather(ci, b):
            pltpu.async_copy(x_hbm.at[src_v.at[ci]], rows_v.at[b], gsems[b])

        def scatter(ci, b):
            pltpu.async_copy(rows_v.at[b], acc.at[dst_v.at[ci]], ssems[b],
                             add=True)

        def drain(sem, b):
            pltpu.make_async_copy(x_hbm.at[pl.ds(0, _CHUNK)],
                                  rows_v.at[b], sem).wait()

        def compute(ci, b):
            def group_body(g, _):
                wv = w_v[ci, pl.ds(g * 16, 16)]
                for j in range(16):
                    wsp = jnp.full((16,), wv[j], jnp.float32)
                    row = g * 16 + j
                    for q in range(d // 16):
                        sl = pl.ds(q * 16, 16)
                        rows_v[b, row, sl] = rows_v[b, row, sl] * wsp
                return 0

            lax.fori_loop(0, ngr, group_body, 0)

        def step(ci, b):
            @pl.when(ci + 1 < cpt)
            def _():
                @pl.when(ci >= 1)
                def _():
                    drain(ssems[1 - b], 1 - b)
                gather(ci + 1, 1 - b)
            drain(gsems[b], b)
            compute(ci, b)
            scatter(ci, b)

        gather(jnp.int32(0), 0)

        def outer(k, _):
            step(2 * k, 0)
            step(2 * k + 1, 1)
            return 0

        lax.fori_loop(0, cpt // 2, outer, 0)
        drain(ssems[0], 0)
        drain(ssems[1], 1)
        plsc.subcore_barrier()

        @pl.when(c == 0)
        def _():
            pltpu.sync_copy(acc.at[pl.ds(r0, rows_pt)], out0.at[pl.ds(r0, rows_pt)])

        @pl.when(c == 1)
        def _():
            pltpu.sync_copy(acc.at[pl.ds(r0, rows_pt)], out1.at[pl.ds(r0, rows_pt)])

    return segsum


def _row_grid(n, d, n_in, body, out_shape=None, block_rows=2000):
    grid = (pl.cdiv(n, block_rows),)
    in_specs = [pl.BlockSpec((block_rows, d), lambda i: (i, 0))] * n_in
    if out_shape is None:
        out_shape = (n, d)
    out_spec = pl.BlockSpec((block_rows, out_shape[1]), lambda i: (i, 0))
    return pl.pallas_call(
        body,
        grid=grid,
        in_specs=in_specs,
        out_specs=out_spec,
        out_shape=jax.ShapeDtypeStruct(out_shape, jnp.float32),
    )


def _x1_body(x0, a00, a01, o):
    o[...] = x0[...] + (a00[...] + a01[...]) * jnp.float32(1.0 / 3.0)


def _x2_body(x0, a00, a01, a10, a11, o):
    o[...] = x0[...] + (a10[...] + a11[...]) - (a00[...] + a01[...]) * jnp.float32(1.0 / 3.0)


def _g_body(x0, x1, x2, a00, a01, a10, a11, a20, a21, o):
    x3 = (x0[...] + (a00[...] + a01[...]) - (a10[...] + a11[...])
          + (a20[...] + a21[...]))
    d = x0.shape[1]
    o[:, 0 * d:1 * d] = x0[...] * jnp.float32(_W0)
    o[:, 1 * d:2 * d] = x1[...] * jnp.float32(_W1)
    o[:, 2 * d:3 * d] = x2[...] * jnp.float32(_W1)
    o[:, 3 * d:4 * d] = x3 * jnp.float32(_W0)


def _mm_body(gi, gj, o):
    o[...] = lax.dot_general(
        gi[...], gj[...], (((1,), (1,)), ((), ())),
        preferred_element_type=jnp.float32,
    )


@functools.lru_cache(maxsize=None)
def _make_gram(n, k, bm, bn):
    grid = (pl.cdiv(n, bm), pl.cdiv(n, bn))
    return pl.pallas_call(
        _mm_body,
        grid=grid,
        in_specs=[
            pl.BlockSpec((bm, k), lambda i, j: (i, 0)),
            pl.BlockSpec((bn, k), lambda i, j: (j, 0)),
        ],
        out_specs=pl.BlockSpec((bm, bn), lambda i, j: (i, j)),
        out_shape=jax.ShapeDtypeStruct((n, n), jnp.float32),
        compiler_params=pltpu.CompilerParams(
            dimension_semantics=("parallel", "parallel"),
        ),
    )


def kernel(h, e, edge_index):
    n, d = h.shape
    ev = e.shape[0]
    src = edge_index[0]
    dst = edge_index[1]
    ew = e.reshape(ev)
    np_ = ((n + NS * 8 - 1) // (NS * 8)) * (NS * 8)
    zeros = jnp.zeros((np_, d), jnp.float32)

    tch = -(-ev // _CHUNK)
    tch = -(-tch // NW) * NW
    if tch // NW % 2:
        tch += NW
    evp = tch * _CHUNK
    if evp > ev:
        padn = evp - ev
        pad_dst = jnp.arange(padn, dtype=jnp.int32) % n
        pad_src = (pad_dst + 1) % n
        src = jnp.concatenate([src, pad_src])
        dst = jnp.concatenate([dst, pad_dst])
        ew = jnp.concatenate([ew, jnp.zeros((padn,), jnp.float32)])
    src2 = src.reshape(tch, _CHUNK)
    dst2 = dst.reshape(tch, _CHUNK)
    ew2 = ew.reshape(tch, _CHUNK)

    segsum = _make_segsum(n, tch, d)
    a00, a01 = segsum(h, src2, dst2, ew2, zeros)
    x1 = _row_grid(n, d, 3, _x1_body)(h, a00, a01)
    a10, a11 = segsum(x1, src2, dst2, ew2, zeros)
    x2 = _row_grid(n, d, 5, _x2_body)(h, a00, a01, a10, a11)
    a20, a21 = segsum(x2, src2, dst2, ew2, zeros)
    g = _row_grid(n, d, 9, _g_body, out_shape=(n, 4 * d))(
        h, x1, x2, a00, a01, a10, a11, a20, a21)
    return _make_gram(n, 4 * d, 1024, 1024)(g, g)


# speedup vs baseline: 8.3570x; 1.0692x over previous
import functools

import jax
import jax.numpy as jnp
from jax import lax
from jax.experimental import pallas as pl
from jax.experimental.pallas import tpu as pltpu
from jax.experimental.pallas import tpu_sc as plsc

NC = 2
NS = 16
NW = NC * NS

_W0 = 0.3535533905932738
_W1 = 0.6123724356957945

_CHUNK = 128


@functools.lru_cache(maxsize=None)
def _make_segsum(n, tch, d):
    assert tch % NW == 0
    cpt = tch // NW
    np_ = ((n + NS * 8 - 1) // (NS * 8)) * (NS * 8)
    rows_pt = np_ // NS
    assert d % 16 == 0
    ngr = _CHUNK // 16

    mesh = plsc.VectorSubcoreMesh(core_axis_name="c", subcore_axis_name="s")

    @functools.partial(
        pl.kernel,
        out_type=[
            jax.ShapeDtypeStruct((np_, d), jnp.float32),
            jax.ShapeDtypeStruct((np_, d), jnp.float32),
        ],
        mesh=mesh,
        scratch_types=[
            pltpu.VMEM_SHARED((np_, d), jnp.float32),
            pltpu.VMEM((cpt, _CHUNK), jnp.int32),
            pltpu.VMEM((cpt, _CHUNK), jnp.int32),
            pltpu.VMEM((cpt, _CHUNK), jnp.float32),
            pltpu.VMEM((2, _CHUNK, d), jnp.float32),
            pltpu.SemaphoreType.DMA,
            pltpu.SemaphoreType.DMA,
            pltpu.SemaphoreType.DMA,
            pltpu.SemaphoreType.DMA,
        ],
    )
    def segsum(x_hbm, src_hbm, dst_hbm, e_hbm, z_hbm,
               out0, out1, acc, src_v, dst_v, w_v, rows_v,
               gsem0, gsem1, ssem0, ssem1):
        c = lax.axis_index("c")
        s = lax.axis_index("s")
        wid = s * NC + c
        r0 = s * rows_pt
        pltpu.sync_copy(z_hbm.at[pl.ds(r0, rows_pt)], acc.at[pl.ds(r0, rows_pt)])
        cb = wid * cpt
        pltpu.sync_copy(src_hbm.at[pl.ds(cb, cpt)], src_v)
        pltpu.sync_copy(dst_hbm.at[pl.ds(cb, cpt)], dst_v)
        pltpu.sync_copy(e_hbm.at[pl.ds(cb, cpt)], w_v)

        def wbody(ci, _):
            for g in range(ngr):
                sl = pl.ds(g * 16, 16)
                w_v[ci, sl] = jnp.where(src_v[ci, sl] == dst_v[ci, sl],
                                        jnp.float32(-1.0), w_v[ci, sl])
            return 0

        lax.fori_loop(0, cpt, wbody, 0)
        plsc.subcore_barrier()

        gsems = (gsem0, gsem1)
        ssems = (ssem0, ssem1)

        def gather(ci, b):
            pltpu.async_copy(x_hbm.at[src_v.at[ci]], rows_v.at[b], gsems[b])

        def scatter(ci, b):
            pltpu.async_copy(rows_v.at[b], acc.at[dst_v.at[ci]], ssems[b],
                             add=True)

        def drain(sem, b):
            pltpu.make_async_copy(x_hbm.at[pl.ds(0, _CHUNK)],
                                  rows_v.at[b], sem).wait()

        def compute(ci, b):
            def group_body(g, _):
                wv = w_v[ci, pl.ds(g * 16, 16)]
                for j in range(16):
                    wsp = jnp.full((16,), wv[j], jnp.float32)
                    row = g * 16 + j
                    for q in range(d // 16):
                        sl = pl.ds(q * 16, 16)
                        rows_v[b, row, sl] = rows_v[b, row, sl] * wsp
                return 0

            lax.fori_loop(0, ngr, group_body, 0)

        def step(ci, b):
            @pl.when(ci + 1 < cpt)
            def _():
                @pl.when(ci >= 1)
                def _():
                    drain(ssems[1 - b], 1 - b)
                gather(ci + 1, 1 - b)
            drain(gsems[b], b)
            compute(ci, b)
            scatter(ci, b)

        gather(jnp.int32(0), 0)

        def outer(k, _):
            step(2 * k, 0)
            step(2 * k + 1, 1)
            return 0

        lax.fori_loop(0, cpt // 2, outer, 0)
        drain(ssems[0], 0)
        drain(ssems[1], 1)
        plsc.subcore_barrier()

        @pl.when(c == 0)
        def _():
            pltpu.sync_copy(acc.at[pl.ds(r0, rows_pt)], out0.at[pl.ds(r0, rows_pt)])

        @pl.when(c == 1)
        def _():
            pltpu.sync_copy(acc.at[pl.ds(r0, rows_pt)], out1.at[pl.ds(r0, rows_pt)])

    return segsum


def _row_grid(n, d, n_in, body, out_shape=None, block_rows=2000,
              out_dtype=jnp.float32):
    grid = (pl.cdiv(n, block_rows),)
    in_specs = [pl.BlockSpec((block_rows, d), lambda i: (i, 0))] * n_in
    if out_shape is None:
        out_shape = (n, d)
    out_spec = pl.BlockSpec((block_rows, out_shape[1]), lambda i: (i, 0))
    return pl.pallas_call(
        body,
        grid=grid,
        in_specs=in_specs,
        out_specs=out_spec,
        out_shape=jax.ShapeDtypeStruct(out_shape, out_dtype),
    )


def _x1_body(x0, a00, a01, o):
    o[...] = x0[...] + (a00[...] + a01[...]) * jnp.float32(1.0 / 3.0)


def _x2_body(x0, a00, a01, a10, a11, o):
    o[...] = x0[...] + (a10[...] + a11[...]) - (a00[...] + a01[...]) * jnp.float32(1.0 / 3.0)


def _g_body(x0, x1, x2, a00, a01, a10, a11, a20, a21, o):
    x3 = (x0[...] + (a00[...] + a01[...]) - (a10[...] + a11[...])
          + (a20[...] + a21[...]))
    d = x0.shape[1]
    o[:, 0 * d:1 * d] = (x0[...] * jnp.float32(_W0)).astype(o.dtype)
    o[:, 1 * d:2 * d] = (x1[...] * jnp.float32(_W1)).astype(o.dtype)
    o[:, 2 * d:3 * d] = (x2[...] * jnp.float32(_W1)).astype(o.dtype)
    o[:, 3 * d:4 * d] = (x3 * jnp.float32(_W0)).astype(o.dtype)


def _mm_body(gi, gj, o):
    o[...] = lax.dot_general(
        gi[...], gj[...], (((1,), (1,)), ((), ())),
        preferred_element_type=jnp.float32,
    )


@functools.lru_cache(maxsize=None)
def _make_gram(n, k, bm, bn):
    grid = (pl.cdiv(n, bm), pl.cdiv(n, bn))
    return pl.pallas_call(
        _mm_body,
        grid=grid,
        in_specs=[
            pl.BlockSpec((bm, k), lambda i, j: (i, 0)),
            pl.BlockSpec((bn, k), lambda i, j: (j, 0)),
        ],
        out_specs=pl.BlockSpec((bm, bn), lambda i, j: (i, j)),
        out_shape=jax.ShapeDtypeStruct((n, n), jnp.float32),
        compiler_params=pltpu.CompilerParams(
            dimension_semantics=("parallel", "parallel"),
        ),
    )


def kernel(h, e, edge_index):
    n, d = h.shape
    ev = e.shape[0]
    src = edge_index[0]
    dst = edge_index[1]
    ew = e.reshape(ev)
    np_ = ((n + NS * 8 - 1) // (NS * 8)) * (NS * 8)
    zeros = jnp.zeros((np_, d), jnp.float32)

    tch = -(-ev // _CHUNK)
    tch = -(-tch // NW) * NW
    if tch // NW % 2:
        tch += NW
    evp = tch * _CHUNK
    if evp > ev:
        padn = evp - ev
        pad_dst = jnp.arange(padn, dtype=jnp.int32) % n
        pad_src = (pad_dst + 1) % n
        src = jnp.concatenate([src, pad_src])
        dst = jnp.concatenate([dst, pad_dst])
        ew = jnp.concatenate([ew, jnp.zeros((padn,), jnp.float32)])
    src2 = src.reshape(tch, _CHUNK)
    dst2 = dst.reshape(tch, _CHUNK)
    ew2 = ew.reshape(tch, _CHUNK)

    segsum = _make_segsum(n, tch, d)
    a00, a01 = segsum(h, src2, dst2, ew2, zeros)
    x1 = _row_grid(n, d, 3, _x1_body)(h, a00, a01)
    a10, a11 = segsum(x1, src2, dst2, ew2, zeros)
    x2 = _row_grid(n, d, 5, _x2_body)(h, a00, a01, a10, a11)
    a20, a21 = segsum(x2, src2, dst2, ew2, zeros)
    g = _row_grid(n, d, 9, _g_body, out_shape=(n, 4 * d),
                  out_dtype=jnp.bfloat16)(
        h, x1, x2, a00, a01, a10, a11, a20, a21)
    return _make_gram(n, 4 * d, 1024, 1024)(g, g)


# speedup vs baseline: 9.4320x; 1.1286x over previous
import functools

import jax
import jax.numpy as jnp
from jax import lax
from jax.experimental import pallas as pl
from jax.experimental.pallas import tpu as pltpu
from jax.experimental.pallas import tpu_sc as plsc

NC = 2
NS = 16
NW = NC * NS

_W0 = 0.3535533905932738
_W1 = 0.6123724356957945

_CHUNK = 128


@functools.lru_cache(maxsize=None)
def _make_segsum(n, tch, d):
    assert tch % NW == 0
    cpt = tch // NW
    np_ = ((n + NS * 8 - 1) // (NS * 8)) * (NS * 8)
    rows_pt = np_ // NS
    assert d % 16 == 0
    ngr = _CHUNK // 16

    mesh = plsc.VectorSubcoreMesh(core_axis_name="c", subcore_axis_name="s")

    @functools.partial(
        pl.kernel,
        out_type=[
            jax.ShapeDtypeStruct((np_, d), jnp.float32),
            jax.ShapeDtypeStruct((np_, d), jnp.float32),
        ],
        mesh=mesh,
        scratch_types=[
            pltpu.VMEM_SHARED((np_, d), jnp.float32),
            pltpu.VMEM((cpt, _CHUNK), jnp.int32),
            pltpu.VMEM((cpt, _CHUNK), jnp.int32),
            pltpu.VMEM((cpt, _CHUNK), jnp.float32),
            pltpu.VMEM((2, _CHUNK, d), jnp.float32),
            pltpu.SemaphoreType.DMA,
            pltpu.SemaphoreType.DMA,
            pltpu.SemaphoreType.DMA,
            pltpu.SemaphoreType.DMA,
        ],
    )
    def segsum(x_hbm, src_hbm, dst_hbm, e_hbm, z_hbm,
               out0, out1, acc, src_v, dst_v, w_v, rows_v,
               gsem0, gsem1, ssem0, ssem1):
        c = lax.axis_index("c")
        s = lax.axis_index("s")
        wid = s * NC + c
        r0 = s * rows_pt
        pltpu.sync_copy(z_hbm.at[pl.ds(r0, rows_pt)], acc.at[pl.ds(r0, rows_pt)])
        cb = wid * cpt
        pltpu.sync_copy(src_hbm.at[pl.ds(cb, cpt)], src_v)
        pltpu.sync_copy(dst_hbm.at[pl.ds(cb, cpt)], dst_v)
        pltpu.sync_copy(e_hbm.at[pl.ds(cb, cpt)], w_v)

        def wbody(ci, _):
            for g in range(ngr):
                sl = pl.ds(g * 16, 16)
                w_v[ci, sl] = jnp.where(src_v[ci, sl] == dst_v[ci, sl],
                                        jnp.float32(-1.0), w_v[ci, sl])
            return 0

        lax.fori_loop(0, cpt, wbody, 0)
        plsc.subcore_barrier()

        gsems = (gsem0, gsem1)
        ssems = (ssem0, ssem1)

        def gather(ci, b):
            pltpu.async_copy(x_hbm.at[src_v.at[ci]], rows_v.at[b], gsems[b])

        def scatter(ci, b):
            pltpu.async_copy(rows_v.at[b], acc.at[dst_v.at[ci]], ssems[b],
                             add=True)

        def drain(sem, b):
            pltpu.make_async_copy(x_hbm.at[pl.ds(0, _CHUNK)],
                                  rows_v.at[b], sem).wait()

        def compute(ci, b):
            def group_body(g, _):
                wv = w_v[ci, pl.ds(g * 16, 16)]
                for j in range(16):
                    wsp = jnp.full((16,), wv[j], jnp.float32)
                    row = g * 16 + j
                    for q in range(d // 16):
                        sl = pl.ds(q * 16, 16)
                        rows_v[b, row, sl] = rows_v[b, row, sl] * wsp
                return 0

            lax.fori_loop(0, ngr, group_body, 0)

        def step(ci, b):
            @pl.when(ci + 1 < cpt)
            def _():
                @pl.when(ci >= 1)
                def _():
                    drain(ssems[1 - b], 1 - b)
                gather(ci + 1, 1 - b)
            drain(gsems[b], b)
            compute(ci, b)
            scatter(ci, b)

        gather(jnp.int32(0), 0)

        def outer(k, _):
            step(2 * k, 0)
            step(2 * k + 1, 1)
            return 0

        lax.fori_loop(0, cpt // 2, outer, 0)
        drain(ssems[0], 0)
        drain(ssems[1], 1)
        plsc.subcore_barrier()

        @pl.when(c == 0)
        def _():
            pltpu.sync_copy(acc.at[pl.ds(r0, rows_pt)], out0.at[pl.ds(r0, rows_pt)])

        @pl.when(c == 1)
        def _():
            pltpu.sync_copy(acc.at[pl.ds(r0, rows_pt)], out1.at[pl.ds(r0, rows_pt)])

    return segsum


def _row_grid(n, d, n_in, body, out_shape=None, block_rows=2000,
              out_dtype=jnp.float32):
    grid = (pl.cdiv(n, block_rows),)
    in_specs = [pl.BlockSpec((block_rows, d), lambda i: (i, 0))] * n_in
    if out_shape is None:
        out_shape = (n, d)
    out_spec = pl.BlockSpec((block_rows, out_shape[1]), lambda i: (i, 0))
    return pl.pallas_call(
        body,
        grid=grid,
        in_specs=in_specs,
        out_specs=out_spec,
        out_shape=jax.ShapeDtypeStruct(out_shape, out_dtype),
    )


def _x1_body(x0, a00, a01, o):
    o[...] = x0[...] + (a00[...] + a01[...]) * jnp.float32(1.0 / 3.0)


def _x2_body(x0, a00, a01, a10, a11, o):
    o[...] = x0[...] + (a10[...] + a11[...]) - (a00[...] + a01[...]) * jnp.float32(1.0 / 3.0)


def _g_body(x0, x1, x2, a00, a01, a10, a11, a20, a21, o):
    x3 = (x0[...] + (a00[...] + a01[...]) - (a10[...] + a11[...])
          + (a20[...] + a21[...]))
    d = x0.shape[1]
    o[:, 0 * d:1 * d] = (x0[...] * jnp.float32(_W0)).astype(o.dtype)
    o[:, 1 * d:2 * d] = (x1[...] * jnp.float32(_W1)).astype(o.dtype)
    o[:, 2 * d:3 * d] = (x2[...] * jnp.float32(_W1)).astype(o.dtype)
    o[:, 3 * d:4 * d] = (x3 * jnp.float32(_W0)).astype(o.dtype)


def _mm_body(gi, gj, o):
    o[...] = lax.dot_general(
        gi[...], gj[...], (((1,), (1,)), ((), ())),
        preferred_element_type=jnp.float32,
    )


@functools.lru_cache(maxsize=None)
def _make_gram(n, k, bm, bn):
    grid = (pl.cdiv(n, bm), pl.cdiv(n, bn))
    return pl.pallas_call(
        _mm_body,
        grid=grid,
        in_specs=[
            pl.BlockSpec((bm, k), lambda i, j: (i, 0)),
            pl.BlockSpec((bn, k), lambda i, j: (j, 0)),
        ],
        out_specs=pl.BlockSpec((bm, bn), lambda i, j: (i, j)),
        out_shape=jax.ShapeDtypeStruct((n, n), jnp.float32),
        compiler_params=pltpu.CompilerParams(
            dimension_semantics=("parallel", "parallel"),
        ),
    )


def kernel(h, e, edge_index):
    n, d = h.shape
    ev = e.shape[0]
    src = edge_index[0]
    dst = edge_index[1]
    ew = e.reshape(ev)
    np_ = ((n + NS * 8 - 1) // (NS * 8)) * (NS * 8)
    zeros = jnp.zeros((np_, d), jnp.float32)

    tch = -(-ev // _CHUNK)
    tch = -(-tch // NW) * NW
    if tch // NW % 2:
        tch += NW
    evp = tch * _CHUNK
    if evp > ev:
        padn = evp - ev
        pad_dst = jnp.arange(padn, dtype=jnp.int32) % n
        pad_src = (pad_dst + 1) % n
        src = jnp.concatenate([src, pad_src])
        dst = jnp.concatenate([dst, pad_dst])
        ew = jnp.concatenate([ew, jnp.zeros((padn,), jnp.float32)])
    src2 = src.reshape(tch, _CHUNK)
    dst2 = dst.reshape(tch, _CHUNK)
    ew2 = ew.reshape(tch, _CHUNK)

    segsum = _make_segsum(n, tch, d)
    a00, a01 = segsum(h, src2, dst2, ew2, zeros)
    x1 = _row_grid(n, d, 3, _x1_body)(h, a00, a01)
    a10, a11 = segsum(x1, src2, dst2, ew2, zeros)
    x2 = _row_grid(n, d, 5, _x2_body)(h, a00, a01, a10, a11)
    a20, a21 = segsum(x2, src2, dst2, ew2, zeros)
    g = _row_grid(n, d, 9, _g_body, out_shape=(n, 4 * d),
                  out_dtype=jnp.bfloat16)(
        h, x1, x2, a00, a01, a10, a11, a20, a21)
    return _make_gram(n, 4 * d, 2048, 2048)(g, g)
